# Initial kernel scaffold; baseline (speedup 1.0000x reference)
import jax
import jax.numpy as jnp
from jax.experimental import pallas as pl


def kernel(x, table, w, W_out, b_out):
    raise NotImplementedError("write your pallas kernel here")


# speedup vs baseline: 27.2939x; 27.2939x over previous
import functools

import jax
import jax.numpy as jnp
from jax import lax
from jax.experimental import pallas as pl
from jax.experimental.pallas import tpu as pltpu
from jax.experimental.pallas import tpu_sc as plsc


def _sc_pooled_embedding(xa, xb, table, w2, B, D, LA, LB, n_workers):
    LP = LA + LB
    b_per_w = B // n_workers
    mesh = plsc.VectorSubcoreMesh(core_axis_name="c", subcore_axis_name="s")

    @functools.partial(
        pl.kernel,
        out_type=jax.ShapeDtypeStruct((B, D), jnp.float32),
        mesh=mesh,
        compiler_params=pltpu.CompilerParams(use_tc_tiling_on_sc=False),
        scratch_types=[
            pltpu.VMEM((LA,), jnp.int32),
            pltpu.VMEM((LB,), jnp.int32),
            pltpu.VMEM((LP,), jnp.float32),
            pltpu.VMEM((LP, D), jnp.float32),
            pltpu.VMEM((b_per_w, D), jnp.float32),
        ],
    )
    def body(xa_hbm, xb_hbm, table_hbm, w_hbm, out_hbm,
             xa_v, xb_v, wv_v, rows_v, out_v):
        wid = lax.axis_index("c") * (n_workers // 2) + lax.axis_index("s")
        base = wid * b_per_w

        def do_row(b, carry):
            gb = base + b
            pltpu.sync_copy(xa_hbm.at[gb], xa_v)
            pltpu.sync_copy(xb_hbm.at[gb], xb_v)
            pltpu.sync_copy(table_hbm.at[xa_v], rows_v.at[pl.ds(0, LA)])
            pltpu.sync_copy(table_hbm.at[xb_v], rows_v.at[pl.ds(LA, LB)])
            pltpu.sync_copy(w_hbm.at[xa_v], wv_v.at[pl.ds(0, LA)])
            pltpu.sync_copy(w_hbm.at[xb_v], wv_v.at[pl.ds(LA, LB)])

            cnt = jnp.zeros((16,), jnp.float32)
            wgs = []
            for i in range(LP // 16):
                off = i * 16
                if off < LA:
                    xv = xa_v[pl.ds(off, 16)]
                else:
                    xv = xb_v[pl.ds(off - LA, 16)]
                m = xv != 0
                wvv = wv_v[pl.ds(off, 16)]
                wgs.append(jnp.where(m, wvv, 0.0))
                cnt = cnt + jnp.where(m, 1.0, 0.0)
            ii = lax.iota(jnp.int32, 16)
            for sh in (8, 4, 2, 1):
                cnt = cnt + cnt.at[ii ^ sh].get(mode="promise_in_bounds")

            accs = [jnp.zeros((16,), jnp.float32) for _ in range(D // 16)]
            for g in range(LP // 16):
                wgv = wgs[g]
                for t in range(16):
                    l = g * 16 + t
                    s = wgv.at[jnp.full((16,), t, jnp.int32)].get(
                        mode="promise_in_bounds")
                    for j in range(D // 16):
                        accs[j] = accs[j] + s * rows_v[l, pl.ds(j * 16, 16)]
            inv = 1.0 / (cnt.astype(jnp.float32) + 1e-16)
            for j in range(D // 16):
                out_v[b, pl.ds(j * 16, 16)] = accs[j] * inv
            return carry

        lax.fori_loop(0, b_per_w, do_row, 0)
        pltpu.sync_copy(out_v, out_hbm.at[pl.ds(base, b_per_w)])

    return body(xa, xb, table, w2)


def _tc_linear(embedded, wt, b2, B, D, OUT):
    BM = 512

    def mm_body(e_ref, w_ref, b_ref, o_ref):
        o_ref[...] = (
            jnp.dot(e_ref[...], w_ref[...], preferred_element_type=jnp.float32)
            + b_ref[...]
        )

    return pl.pallas_call(
        mm_body,
        grid=(B // BM,),
        in_specs=[
            pl.BlockSpec((BM, D), lambda i: (i, 0)),
            pl.BlockSpec((D, OUT), lambda i: (0, 0)),
            pl.BlockSpec((1, OUT), lambda i: (0, 0)),
        ],
        out_specs=pl.BlockSpec((BM, OUT), lambda i: (i, 0)),
        out_shape=jax.ShapeDtypeStruct((B, OUT), jnp.float32),
    )(embedded, wt, b2)


def kernel(x, table, w, W_out, b_out):
    B, L = x.shape
    V, D = table.shape
    OUT = W_out.shape[0]
    N_WORKERS = 32
    LA = 128
    LB = ((L - LA + 15) // 16) * 16
    LP = LA + LB

    x = x.astype(jnp.int32)
    xp = jnp.pad(x, ((0, 0), (0, LP - L)))
    xa = xp[:, :LA]
    xb = xp[:, LA:]

    embedded = _sc_pooled_embedding(xa, xb, table, w, B, D, LA, LB, N_WORKERS)
    out = _tc_linear(embedded, W_out.T, b_out[None, :], B, D, OUT)
    return (out, embedded)


# speedup vs baseline: 31.1380x; 1.1408x over previous
import functools

import jax
import jax.numpy as jnp
from jax import lax
from jax.experimental import pallas as pl
from jax.experimental.pallas import tpu as pltpu
from jax.experimental.pallas import tpu_sc as plsc

_N_WORKERS = 32
_ROWS_PER_TILE = 8


def _sc_pooled_embedding(xf, table, w, B, D, LP):
    RT = _ROWS_PER_TILE
    TOK = RT * LP
    NCH = TOK // 128
    b_per_w = B // _N_WORKERS
    tiles_per_w = b_per_w // RT
    mesh = plsc.VectorSubcoreMesh(core_axis_name="c", subcore_axis_name="s")

    @functools.partial(
        pl.kernel,
        out_type=jax.ShapeDtypeStruct((B, D), jnp.float32),
        mesh=mesh,
        compiler_params=pltpu.CompilerParams(use_tc_tiling_on_sc=False),
        scratch_types=[
            pltpu.VMEM((2, TOK), jnp.int32),
            pltpu.VMEM((2, TOK), jnp.float32),
            pltpu.VMEM((2, TOK, D), jnp.float32),
            pltpu.VMEM((b_per_w, D), jnp.float32),
            pltpu.SemaphoreType.DMA,
            pltpu.SemaphoreType.DMA,
            pltpu.SemaphoreType.DMA,
        ],
    )
    def body(xf_hbm, table_hbm, w_hbm, out_hbm,
             xf_v, wv_v, rows_v, out_v, gsem0, gsem1, xsem):
        wid = lax.axis_index("c") * (_N_WORKERS // 2) + lax.axis_index("s")
        tbase = wid * tiles_per_w

        def descs(par, sem):
            cps = []
            for j in range(NCH):
                idx = xf_v.at[par, pl.ds(j * 128, 128)]
                cps.append(pltpu.make_async_copy(
                    table_hbm.at[idx],
                    rows_v.at[par, pl.ds(j * 128, 128)], sem))
                cps.append(pltpu.make_async_copy(
                    w_hbm.at[idx],
                    wv_v.at[par, pl.ds(j * 128, 128)], sem))
            return cps

        def fire(t, par, sem):
            pltpu.sync_copy(xf_hbm.at[tbase + t], xf_v.at[par])
            for c in descs(par, sem):
                c.start()

        def drain(par, sem):
            for c in descs(par, sem):
                c.wait()

        fire(0, 0, gsem0)

        def do_tile(t, carry):
            par = lax.rem(t, 2)

            @pl.when(t + 1 < tiles_per_w)
            def _():
                lax.cond(par == 0,
                         lambda: fire(t + 1, 1, gsem1),
                         lambda: fire(t + 1, 0, gsem0))

            lax.cond(par == 0,
                     lambda: drain(0, gsem0),
                     lambda: drain(1, gsem1))

            ii = lax.iota(jnp.int32, 16)
            for r in range(RT):
                base = r * LP
                cnt = jnp.zeros((16,), jnp.float32)
                wgs = []
                for i in range(LP // 16):
                    off = base + i * 16
                    xv = xf_v[par, pl.ds(off, 16)]
                    m = xv != 0
                    wvv = wv_v[par, pl.ds(off, 16)]
                    wgs.append(jnp.where(m, wvv, 0.0))
                    cnt = cnt + jnp.where(m, 1.0, 0.0)
                for sh in (8, 4, 2, 1):
                    cnt = cnt + cnt.at[ii ^ sh].get(mode="promise_in_bounds")

                accs = [jnp.zeros((16,), jnp.float32) for _ in range(D // 16)]
                for g in range(LP // 16):
                    wgv = wgs[g]
                    for u in range(16):
                        l = base + g * 16 + u
                        s = wgv.at[jnp.full((16,), u, jnp.int32)].get(
                            mode="promise_in_bounds")
                        for j in range(D // 16):
                            accs[j] = accs[j] + s * rows_v[par, l,
                                                          pl.ds(j * 16, 16)]
                inv = 1.0 / (cnt + 1e-16)
                for j in range(D // 16):
                    out_v[t * RT + r, pl.ds(j * 16, 16)] = accs[j] * inv
            return carry

        lax.fori_loop(0, tiles_per_w, do_tile, 0)
        pltpu.sync_copy(out_v, out_hbm.at[pl.ds(wid * b_per_w, b_per_w)])

    return body(xf, table, w)


def _tc_linear(embedded, wt, b2, B, D, OUT):
    BM = 512

    def mm_body(e_ref, w_ref, b_ref, o_ref):
        o_ref[...] = (
            jnp.dot(e_ref[...], w_ref[...], preferred_element_type=jnp.float32)
            + b_ref[...]
        )

    return pl.pallas_call(
        mm_body,
        grid=(B // BM,),
        in_specs=[
            pl.BlockSpec((BM, D), lambda i: (i, 0)),
            pl.BlockSpec((D, OUT), lambda i: (0, 0)),
            pl.BlockSpec((1, OUT), lambda i: (0, 0)),
        ],
        out_specs=pl.BlockSpec((BM, OUT), lambda i: (i, 0)),
        out_shape=jax.ShapeDtypeStruct((B, OUT), jnp.float32),
    )(embedded, wt, b2)


def kernel(x, table, w, W_out, b_out):
    B, L = x.shape
    V, D = table.shape
    OUT = W_out.shape[0]
    LP = ((L + 15) // 16) * 16
    assert (_ROWS_PER_TILE * LP) % 128 == 0

    x = x.astype(jnp.int32)
    xp = jnp.pad(x, ((0, 0), (0, LP - L)))
    xf = xp.reshape(B // _ROWS_PER_TILE, _ROWS_PER_TILE * LP)

    embedded = _sc_pooled_embedding(xf, table, w, B, D, LP)
    out = _tc_linear(embedded, W_out.T, b_out[None, :], B, D, OUT)
    return (out, embedded)
